# baseline (speedup 1.0000x reference)
import functools

import numpy as np
import jax
import jax.numpy as jnp
from jax import lax
from jax.experimental import pallas as pl
from jax.experimental.pallas import tpu as pltpu
from jax.experimental.pallas import tpu_sc as plsc

D = 128
NCLU = 512
DOUT = 256
SC_CORES = 2
SC_SUBCORES = 16
NC4 = NCLU * 4


def _sc_cog(posT, cid):
    B, _, N = posT.shape
    RPW = N // SC_SUBCORES
    SLC = NC4 // SC_SUBCORES
    mesh = plsc.VectorSubcoreMesh(
        core_axis_name="c", subcore_axis_name="s",
        num_cores=SC_CORES, num_subcores=SC_SUBCORES)

    @functools.partial(
        pl.kernel,
        out_type=jax.ShapeDtypeStruct((B, N * 4), jnp.float32),
        mesh=mesh,
        scratch_types=[
            pltpu.VMEM((3, RPW), jnp.float32),
            pltpu.VMEM((RPW,), jnp.int32),
            pltpu.VMEM((16 * NC4,), jnp.float32),
            pltpu.VMEM((NC4,), jnp.float32),
            pltpu.VMEM((SC_SUBCORES, NC4 // SC_SUBCORES), jnp.float32),
            pltpu.VMEM((NC4 // SC_SUBCORES,), jnp.float32),
            pltpu.VMEM((NC4 // SC_SUBCORES,), jnp.float32),
            pltpu.VMEM((NC4,), jnp.float32),
            pltpu.VMEM((RPW * 4,), jnp.float32),
            pltpu.VMEM_SHARED((SC_SUBCORES, NC4), jnp.float32),
            pltpu.VMEM_SHARED((NC4,), jnp.float32),
        ],
        compiler_params=pltpu.CompilerParams(needs_layout_passes=False),
    )
    def kfn(posT_hbm, cid_hbm, cogg_hbm, posv, cidv, bins3f, binsl, partv,
            accv, cogmy, cogv, outv, bins_sh, cog_sh):
        b = lax.axis_index("c")
        s = lax.axis_index("s")
        base = s * RPW
        lane = lax.iota(jnp.int32, 16)
        zero16 = jnp.zeros((16,), jnp.float32)
        ones16 = jnp.ones((16,), jnp.float32)

        pltpu.sync_copy(posT_hbm.at[b, :, pl.ds(base, RPW)], posv)
        pltpu.sync_copy(cid_hbm.at[b, pl.ds(base, RPW)], cidv)

        def _zero(t, _):
            bins3f[pl.ds(t * 16, 16)] = zero16
            return 0
        lax.fori_loop(0, 16 * NC4 // 16, _zero, 0)

        def _scat(kk, _):
            cidk = cidv[pl.ds(kk * 16, 16)]
            fr = lane * NC4 + cidk * 4
            plsc.addupdate_scatter(bins3f, [fr], posv[0, pl.ds(kk * 16, 16)])
            plsc.addupdate_scatter(bins3f, [fr + 1],
                                   posv[1, pl.ds(kk * 16, 16)])
            plsc.addupdate_scatter(bins3f, [fr + 2],
                                   posv[2, pl.ds(kk * 16, 16)])
            plsc.addupdate_scatter(bins3f, [fr + 3], ones16)
            return 0
        lax.fori_loop(0, RPW // 16, _scat, 0)

        def _red(jc, _):
            acc = bins3f[pl.ds(jc * 16, 16)]
            for r in range(1, 16):
                acc = acc + bins3f[pl.ds(r * NC4 + jc * 16, 16)]
            binsl[pl.ds(jc * 16, 16)] = acc
            return 0
        lax.fori_loop(0, NC4 // 16, _red, 0)

        pltpu.sync_copy(binsl, bins_sh.at[s])
        plsc.subcore_barrier()

        SLCc = NC4 // SC_SUBCORES
        pltpu.sync_copy(bins_sh.at[:, pl.ds(s * SLCc, SLCc)], partv)
        divpat = jnp.bitwise_and(lane, -4) + 3
        def _cog(jc, _):
            acc = partv[0, pl.ds(jc * 16, 16)]
            for r in range(1, 16):
                acc = acc + partv[r, pl.ds(jc * 16, 16)]
            accv[pl.ds(jc * 16, 16)] = acc
            dv = plsc.load_gather(accv, [divpat + jc * 16])
            dv = jnp.maximum(dv, 1.0)
            cogmy[pl.ds(jc * 16, 16)] = acc / dv
            return 0
        lax.fori_loop(0, SLCc // 16, _cog, 0)
        pltpu.sync_copy(cogmy, cog_sh.at[pl.ds(s * SLCc, SLCc)])
        plsc.subcore_barrier()

        pltpu.sync_copy(cog_sh, cogv)
        def _gat(kk, _):
            cidk = cidv[pl.ds(kk * 16, 16)]
            fr = cidk * 4
            ro = (lane + kk * 16) * 4
            for comp in range(4):
                g = plsc.load_gather(cogv, [fr + comp])
                plsc.store_scatter(outv, [ro + comp], g)
            return 0
        lax.fori_loop(0, RPW // 16, _gat, 0)
        pltpu.sync_copy(outv, cogg_hbm.at[b, pl.ds(base * 4, RPW * 4)])

    return kfn(posT, cid)


R1 = 1024


def _rows_body(pos_ref, feat_ref, cogg_ref, w1a_ref, b1a_ref, w1b_ref,
               b1b_ref, w2a_ref, b2a_ref, w2b_ref, b2b_ref, wq_ref, wk_ref,
               wv_ref, hpos_ref, q_ref, kv_ref):
    p = pos_ref[0]
    cg = cogg_ref[0][:, :3]
    lp = p - cg
    n = jnp.sqrt(jnp.sum(lp * lp, axis=1, keepdims=True))
    e1 = jnp.concatenate(
        [lp, n, jnp.zeros((R1, 4), jnp.float32)], axis=1)
    f = feat_ref[0]
    a1 = jnp.maximum(
        jnp.dot(e1, w1a_ref[...], preferred_element_type=jnp.float32)
        + b1a_ref[...], 0.0)
    hp = f + jnp.dot(a1, w1b_ref[...],
                     preferred_element_type=jnp.float32) + b1b_ref[...]
    a2 = jnp.maximum(
        jnp.dot(e1, w2a_ref[...], preferred_element_type=jnp.float32)
        + b2a_ref[...], 0.0)
    hg = f + jnp.dot(a2, w2b_ref[...],
                     preferred_element_type=jnp.float32) + b2b_ref[...]
    hpos_ref[0] = hp
    q_ref[0] = (jnp.dot(hg, wq_ref[...], preferred_element_type=jnp.float32)
                * (1.0 / np.sqrt(D))).astype(jnp.bfloat16)
    kv_ref[0] = jnp.concatenate(
        [jnp.dot(hg, wk_ref[...], preferred_element_type=jnp.float32),
         jnp.dot(hp, wv_ref[...], preferred_element_type=jnp.float32)],
        axis=1).astype(jnp.bfloat16)


def _tc_rows(pos, feat, cogg, w1a_p, b1a, w1b, b1b, w2a_p, b2a, w2b, b2b,
             wq, wk, wv):
    B, N, _ = pos.shape
    grid = (B, N // R1)
    row3 = lambda b, i: (b, i, 0)
    cst = lambda b, i: (0, 0)
    wspec = lambda shp: pl.BlockSpec(shp, cst)
    out = pl.pallas_call(
        _rows_body,
        grid=grid,
        in_specs=[
            pl.BlockSpec((1, R1, 3), row3),
            pl.BlockSpec((1, R1, D), row3),
            pl.BlockSpec((1, R1, 4), row3),
            wspec((8, D)), wspec((1, D)), wspec((D, D)), wspec((1, D)),
            wspec((8, D)), wspec((1, D)), wspec((D, D)), wspec((1, D)),
            wspec((D, D)), wspec((D, D)), wspec((D, D)),
        ],
        out_specs=[pl.BlockSpec((1, R1, D), row3)] * 2
        + [pl.BlockSpec((1, R1, 2 * D), row3)],
        out_shape=[jax.ShapeDtypeStruct((B, N, D), jnp.float32),
                   jax.ShapeDtypeStruct((B, N, D), jnp.bfloat16),
                   jax.ShapeDtypeStruct((B, N, 2 * D), jnp.bfloat16)],
    )(pos, feat, cogg, w1a_p, b1a, w1b, b1b, w2a_p, b2a, w2b, b2b, wq, wk, wv)
    return out


RB = 1024
CBK = 256
WSUB = 8
WINC = CBK * WSUB
NCBS = 4096 // CBK


def _attn_body(lo_ref, nw_ref, q_ref, kv0_ref, kv1_ref, kv2_ref, kv3_ref,
               kv4_ref, kv5_ref, kv6_ref, kv7_ref, hpos_ref, rs_ref, wo_ref,
               bo_ref, g_ref, be_ref, wtd_ref, btd_ref, f2_ref, st_ref, m_s,
               l_s, acc_s, st_s):
    b = pl.program_id(0)
    i = pl.program_id(1)
    j = pl.program_id(2)
    nb = nw_ref[b, i]

    @pl.when(j < nb)
    def _():
        q = q_ref[0]
        kv = jnp.concatenate(
            [kv0_ref[0], kv1_ref[0], kv2_ref[0], kv3_ref[0], kv4_ref[0],
             kv5_ref[0], kv6_ref[0], kv7_ref[0]], axis=0)
        kk = kv[:, :D]
        vv = kv[:, D:]
        s = lax.dot_general(q, kk, (((1,), (1,)), ((), ())),
                            preferred_element_type=jnp.float32)
        ilane = jax.lax.broadcasted_iota(jnp.int32, (1, WINC), 1)
        sub = ilane // CBK
        cblk = lo_ref[b, i] + j * WSUB + sub
        colabs = (jnp.minimum(cblk, NCBS - 1) * CBK
                  + (ilane - sub * CBK)).astype(jnp.float32)
        valid = cblk < NCBS
        rs = rs_ref[0]
        ss = rs[:, 0:1]
        se = rs[:, 1:2]
        mask = jnp.logical_and(
            valid, jnp.logical_and(colabs >= ss, colabs < se))
        s = jnp.where(mask, s, -1e9)

        @pl.when(j == 0)
        def _():
            m_s[...] = jnp.full((RB, 128), -1e30, jnp.float32)
            l_s[...] = jnp.zeros((RB, 128), jnp.float32)
            acc_s[...] = jnp.zeros((RB, D), jnp.float32)

        m_curr = jnp.max(s, axis=1, keepdims=True)
        m_prev = m_s[:, :1]
        m_new = jnp.maximum(m_prev, m_curr)
        alpha = jnp.exp(m_prev - m_new)
        p_ = jnp.exp(s - m_new)
        l_new = l_s[:, :1] * alpha + jnp.sum(p_, axis=1, keepdims=True)
        acc_s[...] = acc_s[...] * alpha + lax.dot_general(
            p_.astype(jnp.bfloat16), vv, (((1,), (0,)), ((), ())),
            preferred_element_type=jnp.float32)
        m_s[...] = jnp.broadcast_to(m_new, (RB, 128))
        l_s[...] = jnp.broadcast_to(l_new, (RB, 128))

        @pl.when(j == nb - 1)
        def _():
            o = acc_s[...] / l_s[:, :1]
            o = lax.dot_general(o, wo_ref[...], (((1,), (0,)), ((), ())),
                                preferred_element_type=jnp.float32)
            o = o + bo_ref[...] + hpos_ref[0]
            mu = jnp.mean(o, axis=1, keepdims=True)
            oc = o - mu
            var = jnp.mean(oc * oc, axis=1, keepdims=True)
            ob = oc * lax.rsqrt(var + 1e-5) * g_ref[...] + be_ref[...]
            f2 = lax.dot_general(ob, wtd_ref[...], (((1,), (0,)), ((), ())),
                                 preferred_element_type=jnp.float32)
            f2 = f2 + btd_ref[...]
            f2_ref[0] = f2

            @pl.when(jnp.logical_and(b == 0, i == 0))
            def _():
                st_s[...] = jnp.zeros((8, DOUT), jnp.float32)

            st_s[0:1, :] = st_s[0:1, :] + jnp.sum(f2, axis=0, keepdims=True)
            st_s[1:2, :] = st_s[1:2, :] + jnp.sum(f2 * f2, axis=0,
                                                  keepdims=True)
            st_ref[...] = st_s[...]


def _tc_attn(q, kv, hpos, rowseg, lo, nw, wo, bo, ln1_g, ln1_b, wtd, btd):
    B, N, _ = q.shape
    NR = N // RB
    MAXJ = -(-N // WINC)
    grid = (B, NR, MAXJ)

    def qmap(b, i, j, lo_r, nw_r):
        return (b, i, 0)

    def kvmap(t):
        def _m(b, i, j, lo_r, nw_r):
            jj = jnp.minimum(lo_r[b, i] + j * WSUB + t, NCBS - 1)
            return (b, jj, 0)
        return _m

    def rsmap(b, i, j, lo_r, nw_r):
        return (b * NR + i, 0, 0)

    cst = lambda b, i, j, lo_r, nw_r: (0, 0)
    wspec = lambda shp: pl.BlockSpec(shp, cst)
    grid_spec = pltpu.PrefetchScalarGridSpec(
        num_scalar_prefetch=2,
        grid=grid,
        in_specs=[pl.BlockSpec((1, RB, D), qmap)]
        + [pl.BlockSpec((1, CBK, 2 * D), kvmap(t)) for t in range(WSUB)]
        + [pl.BlockSpec((1, RB, D), qmap),
           pl.BlockSpec((1, RB, 2), rsmap)]
        + [wspec((D, D)), wspec((1, D)), wspec((1, D)), wspec((1, D)),
           wspec((D, DOUT)), wspec((1, DOUT))],
        out_specs=[
            pl.BlockSpec((1, RB, DOUT), qmap),
            pl.BlockSpec((8, DOUT), cst),
        ],
        scratch_shapes=[
            pltpu.VMEM((RB, 128), jnp.float32),
            pltpu.VMEM((RB, 128), jnp.float32),
            pltpu.VMEM((RB, D), jnp.float32),
            pltpu.VMEM((8, DOUT), jnp.float32),
        ],
    )
    f2, st = pl.pallas_call(
        _attn_body,
        grid_spec=grid_spec,
        out_shape=[
            jax.ShapeDtypeStruct((B, N, DOUT), jnp.float32),
            jax.ShapeDtypeStruct((8, DOUT), jnp.float32),
        ],
    )(lo, nw, q, kv, kv, kv, kv, kv, kv, kv, kv, hpos, rowseg, wo, bo,
      ln1_g, ln1_b, wtd, btd)
    return f2, st


def _sc_down(f2f, kidx2, fpsf, posf, scale, shift):
    BN = f2f.shape[0]
    P = fpsf.shape[0]
    NW = SC_CORES * SC_SUBCORES
    PW = P // NW
    NG = PW // 8
    mesh = plsc.VectorSubcoreMesh(
        core_axis_name="c", subcore_axis_name="s",
        num_cores=SC_CORES, num_subcores=SC_SUBCORES)

    @functools.partial(
        pl.kernel,
        out_type=(jax.ShapeDtypeStruct((P * DOUT,), jnp.float32),
                  jax.ShapeDtypeStruct((P * 3,), jnp.float32)),
        mesh=mesh,
        scratch_types=[
            pltpu.VMEM((NG, 128), jnp.int32),
            pltpu.VMEM((128, DOUT), jnp.float32),
            pltpu.VMEM((128, DOUT), jnp.float32),
            pltpu.VMEM((DOUT,), jnp.float32),
            pltpu.VMEM((DOUT,), jnp.float32),
            pltpu.VMEM((PW * DOUT,), jnp.float32),
            pltpu.VMEM((PW,), jnp.int32),
            pltpu.VMEM((BN * 3,), jnp.float32),
            pltpu.VMEM((PW * 3,), jnp.float32),
            pltpu.SemaphoreType.DMA,
            pltpu.SemaphoreType.DMA,
        ],
        compiler_params=pltpu.CompilerParams(needs_layout_passes=False),
    )
    def kfn(f2_hbm, kidx_hbm, fps_hbm, pos_hbm, scale_hbm, shift_hbm,
            featd_hbm, posd_hbm, kidxv, rows0, rows1, scalev, shiftv, outv,
            fpsv, posv, outp, sem0, sem1):
        c = lax.axis_index("c")
        s = lax.axis_index("s")
        wid = c * SC_SUBCORES + s
        lane = lax.iota(jnp.int32, 16)

        pltpu.sync_copy(kidx_hbm.at[pl.ds(wid * NG, NG)], kidxv)
        pltpu.sync_copy(scale_hbm, scalev)
        pltpu.sync_copy(shift_hbm, shiftv)
        pltpu.sync_copy(fps_hbm.at[pl.ds(wid * PW, PW)], fpsv)
        pltpu.sync_copy(pos_hbm, posv)

        def _process(g, rbuf):
            for p in range(8):
                def _chunk(ch, _, p=p):
                    cs = ch * 16
                    mx = rbuf[p * 16, pl.ds(cs, 16)]
                    mn = mx
                    for r in range(1, 16):
                        vv = rbuf[p * 16 + r, pl.ds(cs, 16)]
                        mx = jnp.maximum(mx, vv)
                        mn = jnp.minimum(mn, vv)
                    sc = scalev[pl.ds(cs, 16)]
                    sh = shiftv[pl.ds(cs, 16)]
                    val = jnp.maximum(sc * mx + sh, sc * mn + sh)
                    val = jnp.maximum(val, 0.0)
                    outv[pl.ds((g * 8 + p) * DOUT + cs, 16)] = val
                    return 0
                lax.fori_loop(0, DOUT // 16, _chunk, 0)

        pltpu.async_copy(f2_hbm.at[kidxv.at[0]], rows0, sem0)
        def _outer(t, _):
            g0 = t * 2
            pltpu.make_async_copy(f2_hbm.at[kidxv.at[g0]], rows0, sem0).wait()
            pltpu.async_copy(f2_hbm.at[kidxv.at[g0 + 1]], rows1, sem1)
            _process(g0, rows0)
            pltpu.make_async_copy(
                f2_hbm.at[kidxv.at[g0 + 1]], rows1, sem1).wait()
            @pl.when(t + 1 < NG // 2)
            def _():
                pltpu.async_copy(f2_hbm.at[kidxv.at[g0 + 2]], rows0, sem0)
            _process(g0 + 1, rows1)
            return 0
        lax.fori_loop(0, NG // 2, _outer, 0)
        pltpu.sync_copy(outv, featd_hbm.at[pl.ds(wid * PW * DOUT, PW * DOUT)])

        for jf in range(PW // 16):
            idx = fpsv[pl.ds(jf * 16, 16)]
            ro = (lane + jf * 16) * 3
            fr = idx * 3
            for comp in range(3):
                g = plsc.load_gather(posv, [fr + comp])
                plsc.store_scatter(outp, [ro + comp], g)
        pltpu.sync_copy(outp, posd_hbm.at[pl.ds(wid * PW * 3, PW * 3)])

    return kfn(f2f, kidx2, fpsf, posf, scale, shift)


def kernel(pos, feat, cluster_idx, fps_idx, k_idx, w1a, b1a, w1b, b1b, w2a,
           b2a, w2b, b2b, wq, wk, wv, wo, bo, ln1_g, ln1_b, wtd, btd, bn_g,
           bn_b):
    B, N, _ = pos.shape
    ND = N // 4
    cid = cluster_idx.astype(jnp.int32)
    posT = jnp.transpose(pos, (0, 2, 1))

    cogg = _sc_cog(posT, cid).reshape(B, N, 4)

    w1a_p = jnp.concatenate([w1a, jnp.zeros((4, D), jnp.float32)], axis=0)
    w2a_p = jnp.concatenate([w2a[3:6], jnp.zeros((5, D), jnp.float32)],
                            axis=0)
    hpos, q, kv = _tc_rows(
        pos, feat, cogg, w1a_p, b1a.reshape(1, D), w1b, b1b.reshape(1, D),
        w2a_p, b2a.reshape(1, D), w2b, b2b.reshape(1, D), wq, wk, wv)

    NR = N // RB
    ar = jnp.arange(N, dtype=jnp.int32)[None, :]
    chg_lo = jnp.concatenate(
        [jnp.ones((B, 1), bool), cid[:, 1:] != cid[:, :-1]], axis=1)
    ss = lax.cummax(jnp.where(chg_lo, ar, 0), axis=1)
    endcand = jnp.where(
        jnp.concatenate([chg_lo[:, 1:], jnp.ones((B, 1), bool)], axis=1),
        ar + 1, N)
    se = lax.cummin(endcand[:, ::-1], axis=1)[:, ::-1]
    rowseg = jnp.stack(
        [ss.astype(jnp.float32), se.astype(jnp.float32)],
        axis=-1).reshape(B * NR, RB, 2)
    lo = (ss[:, ::RB] // CBK).astype(jnp.int32)
    nw = ((se[:, RB - 1::RB] - lo * CBK + WINC - 1) // WINC).astype(jnp.int32)

    f2, st = _tc_attn(q, kv, hpos, rowseg, lo, nw, wo,
                      bo.reshape(1, D), ln1_g.reshape(1, D),
                      ln1_b.reshape(1, D), wtd, btd.reshape(1, DOUT))

    cnt = B * N
    m = st[0] / cnt
    var = st[1] / cnt - m * m
    scale = bn_g / jnp.sqrt(var + 1e-5)
    shift = bn_b - m * scale

    boff = (jnp.arange(B, dtype=jnp.int32) * N)[:, None, None]
    kflat = (k_idx.astype(jnp.int32) + boff).reshape(-1, 128)
    fpsf = (fps_idx.astype(jnp.int32) + boff[:, :, 0]).reshape(-1)
    f2f = f2.reshape(B * N, DOUT)
    posf = pos.reshape(-1)

    featd, posd = _sc_down(f2f, kflat, fpsf, posf, scale, shift)
    pos_down = posd.reshape(B, ND, 3)
    feat_down = featd.reshape(B, ND, DOUT)
    return pos_down, feat_down


# speedup vs baseline: 1.1065x; 1.1065x over previous
import functools

import numpy as np
import jax
import jax.numpy as jnp
from jax import lax
from jax.experimental import pallas as pl
from jax.experimental.pallas import tpu as pltpu
from jax.experimental.pallas import tpu_sc as plsc

D = 128
NCLU = 512
DOUT = 256
SC_CORES = 2
SC_SUBCORES = 16
NC4 = NCLU * 4


def _sc_cog(posT, cid):
    B, _, N = posT.shape
    RPW = N // SC_SUBCORES
    SLC = NC4 // SC_SUBCORES
    mesh = plsc.VectorSubcoreMesh(
        core_axis_name="c", subcore_axis_name="s",
        num_cores=SC_CORES, num_subcores=SC_SUBCORES)

    @functools.partial(
        pl.kernel,
        out_type=jax.ShapeDtypeStruct((B, N * 4), jnp.float32),
        mesh=mesh,
        scratch_types=[
            pltpu.VMEM((3, RPW), jnp.float32),
            pltpu.VMEM((RPW,), jnp.int32),
            pltpu.VMEM((16 * NC4,), jnp.float32),
            pltpu.VMEM((NC4,), jnp.float32),
            pltpu.VMEM((SC_SUBCORES, NC4 // SC_SUBCORES), jnp.float32),
            pltpu.VMEM((NC4 // SC_SUBCORES,), jnp.float32),
            pltpu.VMEM((NC4 // SC_SUBCORES,), jnp.float32),
            pltpu.VMEM((NC4,), jnp.float32),
            pltpu.VMEM((RPW * 4,), jnp.float32),
            pltpu.VMEM_SHARED((SC_SUBCORES, NC4), jnp.float32),
            pltpu.VMEM_SHARED((NC4,), jnp.float32),
        ],
        compiler_params=pltpu.CompilerParams(needs_layout_passes=False),
    )
    def kfn(posT_hbm, cid_hbm, cogg_hbm, posv, cidv, bins3f, binsl, partv,
            accv, cogmy, cogv, outv, bins_sh, cog_sh):
        b = lax.axis_index("c")
        s = lax.axis_index("s")
        base = s * RPW
        lane = lax.iota(jnp.int32, 16)
        zero16 = jnp.zeros((16,), jnp.float32)
        ones16 = jnp.ones((16,), jnp.float32)

        pltpu.sync_copy(posT_hbm.at[b, :, pl.ds(base, RPW)], posv)
        pltpu.sync_copy(cid_hbm.at[b, pl.ds(base, RPW)], cidv)

        def _zero(t, _):
            bins3f[pl.ds(t * 16, 16)] = zero16
            return 0
        lax.fori_loop(0, 16 * NC4 // 16, _zero, 0)

        def _scat(kk, _):
            cidk = cidv[pl.ds(kk * 16, 16)]
            fr = lane * NC4 + cidk * 4
            plsc.addupdate_scatter(bins3f, [fr], posv[0, pl.ds(kk * 16, 16)])
            plsc.addupdate_scatter(bins3f, [fr + 1],
                                   posv[1, pl.ds(kk * 16, 16)])
            plsc.addupdate_scatter(bins3f, [fr + 2],
                                   posv[2, pl.ds(kk * 16, 16)])
            plsc.addupdate_scatter(bins3f, [fr + 3], ones16)
            return 0
        lax.fori_loop(0, RPW // 16, _scat, 0)

        def _red(jc, _):
            acc = bins3f[pl.ds(jc * 16, 16)]
            for r in range(1, 16):
                acc = acc + bins3f[pl.ds(r * NC4 + jc * 16, 16)]
            binsl[pl.ds(jc * 16, 16)] = acc
            return 0
        lax.fori_loop(0, NC4 // 16, _red, 0)

        pltpu.sync_copy(binsl, bins_sh.at[s])
        plsc.subcore_barrier()

        SLCc = NC4 // SC_SUBCORES
        pltpu.sync_copy(bins_sh.at[:, pl.ds(s * SLCc, SLCc)], partv)
        divpat = jnp.bitwise_and(lane, -4) + 3
        def _cog(jc, _):
            acc = partv[0, pl.ds(jc * 16, 16)]
            for r in range(1, 16):
                acc = acc + partv[r, pl.ds(jc * 16, 16)]
            accv[pl.ds(jc * 16, 16)] = acc
            dv = plsc.load_gather(accv, [divpat + jc * 16])
            dv = jnp.maximum(dv, 1.0)
            cogmy[pl.ds(jc * 16, 16)] = acc / dv
            return 0
        lax.fori_loop(0, SLCc // 16, _cog, 0)
        pltpu.sync_copy(cogmy, cog_sh.at[pl.ds(s * SLCc, SLCc)])
        plsc.subcore_barrier()

        pltpu.sync_copy(cog_sh, cogv)
        def _gat(kk, _):
            cidk = cidv[pl.ds(kk * 16, 16)]
            fr = cidk * 4
            ro = (lane + kk * 16) * 4
            for comp in range(4):
                g = plsc.load_gather(cogv, [fr + comp])
                plsc.store_scatter(outv, [ro + comp], g)
            return 0
        lax.fori_loop(0, RPW // 16, _gat, 0)
        pltpu.sync_copy(outv, cogg_hbm.at[b, pl.ds(base * 4, RPW * 4)])

    return kfn(posT, cid)


R1 = 1024


def _rows_body(pos_ref, feat_ref, cogg_ref, w1a_ref, b1a_ref, w1b_ref,
               b1b_ref, w2a_ref, b2a_ref, w2b_ref, b2b_ref, wq_ref, wk_ref,
               wv_ref, hpos_ref, q_ref, kv_ref):
    p = pos_ref[0]
    cg = cogg_ref[0][:, :3]
    lp = p - cg
    n = jnp.sqrt(jnp.sum(lp * lp, axis=1, keepdims=True))
    e1 = jnp.concatenate(
        [lp, n, jnp.zeros((R1, 4), jnp.float32)], axis=1)
    f = feat_ref[0]
    a1 = jnp.maximum(
        jnp.dot(e1, w1a_ref[...], preferred_element_type=jnp.float32)
        + b1a_ref[...], 0.0)
    hp = f + jnp.dot(a1, w1b_ref[...],
                     preferred_element_type=jnp.float32) + b1b_ref[...]
    a2 = jnp.maximum(
        jnp.dot(e1, w2a_ref[...], preferred_element_type=jnp.float32)
        + b2a_ref[...], 0.0)
    hg = f + jnp.dot(a2, w2b_ref[...],
                     preferred_element_type=jnp.float32) + b2b_ref[...]
    hpos_ref[0] = hp
    q_ref[0] = (jnp.dot(hg, wq_ref[...], preferred_element_type=jnp.float32)
                * (1.0 / np.sqrt(D))).astype(jnp.bfloat16)
    kv_ref[0] = jnp.concatenate(
        [jnp.dot(hg, wk_ref[...], preferred_element_type=jnp.float32),
         jnp.dot(hp, wv_ref[...], preferred_element_type=jnp.float32)],
        axis=1).astype(jnp.bfloat16)


def _tc_rows(pos, feat, cogg, w1a_p, b1a, w1b, b1b, w2a_p, b2a, w2b, b2b,
             wq, wk, wv):
    B, N, _ = pos.shape
    grid = (B, N // R1)
    row3 = lambda b, i: (b, i, 0)
    cst = lambda b, i: (0, 0)
    wspec = lambda shp: pl.BlockSpec(shp, cst)
    out = pl.pallas_call(
        _rows_body,
        grid=grid,
        in_specs=[
            pl.BlockSpec((1, R1, 3), row3),
            pl.BlockSpec((1, R1, D), row3),
            pl.BlockSpec((1, R1, 4), row3),
            wspec((8, D)), wspec((1, D)), wspec((D, D)), wspec((1, D)),
            wspec((8, D)), wspec((1, D)), wspec((D, D)), wspec((1, D)),
            wspec((D, D)), wspec((D, D)), wspec((D, D)),
        ],
        out_specs=[pl.BlockSpec((1, R1, D), row3)] * 2
        + [pl.BlockSpec((1, R1, 2 * D), row3)],
        out_shape=[jax.ShapeDtypeStruct((B, N, D), jnp.float32),
                   jax.ShapeDtypeStruct((B, N, D), jnp.bfloat16),
                   jax.ShapeDtypeStruct((B, N, 2 * D), jnp.bfloat16)],
    )(pos, feat, cogg, w1a_p, b1a, w1b, b1b, w2a_p, b2a, w2b, b2b, wq, wk, wv)
    return out


RB = 1024
CBK = 256
WSUB = 6
WINC = CBK * WSUB
NCBS = 4096 // CBK


def _attn_body(lo_ref, nw_ref, q_ref, kv0_ref, kv1_ref, kv2_ref, kv3_ref,
               kv4_ref, kv5_ref, hpos_ref, rs_ref, wo_ref, bo_ref, g_ref,
               be_ref, wtd_ref, btd_ref, f2_ref, st_ref, m_s, l_s, acc_s,
               st_s):
    b = pl.program_id(0)
    i = pl.program_id(1)
    j = pl.program_id(2)
    nb = nw_ref[b, i]

    @pl.when(j < nb)
    def _():
        q = q_ref[0]
        kv = jnp.concatenate(
            [kv0_ref[0], kv1_ref[0], kv2_ref[0], kv3_ref[0], kv4_ref[0],
             kv5_ref[0]], axis=0)
        kk = kv[:, :D]
        vv = kv[:, D:]
        s = lax.dot_general(q, kk, (((1,), (1,)), ((), ())),
                            preferred_element_type=jnp.float32)
        ilane = jax.lax.broadcasted_iota(jnp.int32, (1, WINC), 1)
        sub = ilane // CBK
        cblk = lo_ref[b, i] + j * WSUB + sub
        colabs = (jnp.minimum(cblk, NCBS - 1) * CBK
                  + (ilane - sub * CBK)).astype(jnp.float32)
        valid = cblk < NCBS
        rs = rs_ref[0]
        ss = rs[:, 0:1]
        se = rs[:, 1:2]
        mask = jnp.logical_and(
            valid, jnp.logical_and(colabs >= ss, colabs < se))
        s = jnp.where(mask, s, -1e9)

        def _epilogue(o):
            o = lax.dot_general(o, wo_ref[...], (((1,), (0,)), ((), ())),
                                preferred_element_type=jnp.float32)
            o = o + bo_ref[...] + hpos_ref[0]
            mu = jnp.mean(o, axis=1, keepdims=True)
            oc = o - mu
            var = jnp.mean(oc * oc, axis=1, keepdims=True)
            ob = oc * lax.rsqrt(var + 1e-5) * g_ref[...] + be_ref[...]
            f2 = lax.dot_general(ob, wtd_ref[...], (((1,), (0,)), ((), ())),
                                 preferred_element_type=jnp.float32)
            f2 = f2 + btd_ref[...]
            f2_ref[0] = f2

            @pl.when(jnp.logical_and(b == 0, i == 0))
            def _():
                st_s[...] = jnp.zeros((8, DOUT), jnp.float32)

            st_s[0:1, :] = st_s[0:1, :] + jnp.sum(f2, axis=0, keepdims=True)
            st_s[1:2, :] = st_s[1:2, :] + jnp.sum(f2 * f2, axis=0,
                                                  keepdims=True)
            st_ref[...] = st_s[...]

        @pl.when(nb == 1)
        def _():
            m1 = jnp.max(s, axis=1, keepdims=True)
            p1 = jnp.exp(s - m1)
            l1 = jnp.sum(p1, axis=1, keepdims=True)
            acc = lax.dot_general(
                p1.astype(jnp.bfloat16), vv, (((1,), (0,)), ((), ())),
                preferred_element_type=jnp.float32)
            _epilogue(acc / l1)

        @pl.when(nb > 1)
        def _():
            @pl.when(j == 0)
            def _():
                m_s[...] = jnp.full((RB, 128), -1e30, jnp.float32)
                l_s[...] = jnp.zeros((RB, 128), jnp.float32)
                acc_s[...] = jnp.zeros((RB, D), jnp.float32)

            m_curr = jnp.max(s, axis=1, keepdims=True)
            m_prev = m_s[:, :1]
            m_new = jnp.maximum(m_prev, m_curr)
            alpha = jnp.exp(m_prev - m_new)
            p_ = jnp.exp(s - m_new)
            l_new = l_s[:, :1] * alpha + jnp.sum(p_, axis=1, keepdims=True)
            acc_s[...] = acc_s[...] * alpha + lax.dot_general(
                p_.astype(jnp.bfloat16), vv, (((1,), (0,)), ((), ())),
                preferred_element_type=jnp.float32)
            m_s[...] = jnp.broadcast_to(m_new, (RB, 128))
            l_s[...] = jnp.broadcast_to(l_new, (RB, 128))

            @pl.when(j == nb - 1)
            def _():
                _epilogue(acc_s[...] / l_s[:, :1])


def _tc_attn(q, kv, hpos, rowseg, lo, nw, wo, bo, ln1_g, ln1_b, wtd, btd):
    B, N, _ = q.shape
    NR = N // RB
    MAXJ = -(-N // WINC)
    grid = (B, NR, MAXJ)

    def qmap(b, i, j, lo_r, nw_r):
        return (b, i, 0)

    def kvmap(t):
        def _m(b, i, j, lo_r, nw_r):
            jj = jnp.minimum(lo_r[b, i] + j * WSUB + t, NCBS - 1)
            return (b, jj, 0)
        return _m

    def rsmap(b, i, j, lo_r, nw_r):
        return (b * NR + i, 0, 0)

    cst = lambda b, i, j, lo_r, nw_r: (0, 0)
    wspec = lambda shp: pl.BlockSpec(shp, cst)
    grid_spec = pltpu.PrefetchScalarGridSpec(
        num_scalar_prefetch=2,
        grid=grid,
        in_specs=[pl.BlockSpec((1, RB, D), qmap)]
        + [pl.BlockSpec((1, CBK, 2 * D), kvmap(t)) for t in range(WSUB)]
        + [pl.BlockSpec((1, RB, D), qmap),
           pl.BlockSpec((1, RB, 2), rsmap)]
        + [wspec((D, D)), wspec((1, D)), wspec((1, D)), wspec((1, D)),
           wspec((D, DOUT)), wspec((1, DOUT))],
        out_specs=[
            pl.BlockSpec((1, RB, DOUT), qmap),
            pl.BlockSpec((8, DOUT), cst),
        ],
        scratch_shapes=[
            pltpu.VMEM((RB, 128), jnp.float32),
            pltpu.VMEM((RB, 128), jnp.float32),
            pltpu.VMEM((RB, D), jnp.float32),
            pltpu.VMEM((8, DOUT), jnp.float32),
        ],
    )
    f2, st = pl.pallas_call(
        _attn_body,
        grid_spec=grid_spec,
        out_shape=[
            jax.ShapeDtypeStruct((B, N, DOUT), jnp.float32),
            jax.ShapeDtypeStruct((8, DOUT), jnp.float32),
        ],
    )(lo, nw, q, kv, kv, kv, kv, kv, kv, hpos, rowseg, wo, bo, ln1_g, ln1_b,
      wtd, btd)
    return f2, st


def _sc_down(f2f, kidx2, fpsf, posf, scale, shift):
    BN = f2f.shape[0]
    P = fpsf.shape[0]
    NW = SC_CORES * SC_SUBCORES
    PW = P // NW
    NG = PW // 8
    mesh = plsc.VectorSubcoreMesh(
        core_axis_name="c", subcore_axis_name="s",
        num_cores=SC_CORES, num_subcores=SC_SUBCORES)

    @functools.partial(
        pl.kernel,
        out_type=(jax.ShapeDtypeStruct((P * DOUT,), jnp.float32),
                  jax.ShapeDtypeStruct((P * 3,), jnp.float32)),
        mesh=mesh,
        scratch_types=[
            pltpu.VMEM((NG, 128), jnp.int32),
            pltpu.VMEM((128, DOUT), jnp.float32),
            pltpu.VMEM((128, DOUT), jnp.float32),
            pltpu.VMEM((DOUT,), jnp.float32),
            pltpu.VMEM((DOUT,), jnp.float32),
            pltpu.VMEM((PW * DOUT,), jnp.float32),
            pltpu.VMEM((PW,), jnp.int32),
            pltpu.VMEM((BN * 3,), jnp.float32),
            pltpu.VMEM((PW * 3,), jnp.float32),
            pltpu.SemaphoreType.DMA,
            pltpu.SemaphoreType.DMA,
        ],
        compiler_params=pltpu.CompilerParams(needs_layout_passes=False),
    )
    def kfn(f2_hbm, kidx_hbm, fps_hbm, pos_hbm, scale_hbm, shift_hbm,
            featd_hbm, posd_hbm, kidxv, rows0, rows1, scalev, shiftv, outv,
            fpsv, posv, outp, sem0, sem1):
        c = lax.axis_index("c")
        s = lax.axis_index("s")
        wid = c * SC_SUBCORES + s
        lane = lax.iota(jnp.int32, 16)

        pltpu.sync_copy(kidx_hbm.at[pl.ds(wid * NG, NG)], kidxv)
        pltpu.sync_copy(scale_hbm, scalev)
        pltpu.sync_copy(shift_hbm, shiftv)
        pltpu.sync_copy(fps_hbm.at[pl.ds(wid * PW, PW)], fpsv)
        pltpu.sync_copy(pos_hbm, posv)

        def _process(g, rbuf):
            for p in range(8):
                def _chunk(ch, _, p=p):
                    cs = ch * 16
                    mx = rbuf[p * 16, pl.ds(cs, 16)]
                    mn = mx
                    for r in range(1, 16):
                        vv = rbuf[p * 16 + r, pl.ds(cs, 16)]
                        mx = jnp.maximum(mx, vv)
                        mn = jnp.minimum(mn, vv)
                    sc = scalev[pl.ds(cs, 16)]
                    sh = shiftv[pl.ds(cs, 16)]
                    val = jnp.maximum(sc * mx + sh, sc * mn + sh)
                    val = jnp.maximum(val, 0.0)
                    outv[pl.ds((g * 8 + p) * DOUT + cs, 16)] = val
                    return 0
                lax.fori_loop(0, DOUT // 16, _chunk, 0)

        pltpu.async_copy(f2_hbm.at[kidxv.at[0]], rows0, sem0)
        def _outer(t, _):
            g0 = t * 2
            pltpu.make_async_copy(f2_hbm.at[kidxv.at[g0]], rows0, sem0).wait()
            pltpu.async_copy(f2_hbm.at[kidxv.at[g0 + 1]], rows1, sem1)
            _process(g0, rows0)
            pltpu.make_async_copy(
                f2_hbm.at[kidxv.at[g0 + 1]], rows1, sem1).wait()
            @pl.when(t + 1 < NG // 2)
            def _():
                pltpu.async_copy(f2_hbm.at[kidxv.at[g0 + 2]], rows0, sem0)
            _process(g0 + 1, rows1)
            return 0
        lax.fori_loop(0, NG // 2, _outer, 0)
        pltpu.sync_copy(outv, featd_hbm.at[pl.ds(wid * PW * DOUT, PW * DOUT)])

        for jf in range(PW // 16):
            idx = fpsv[pl.ds(jf * 16, 16)]
            ro = (lane + jf * 16) * 3
            fr = idx * 3
            for comp in range(3):
                g = plsc.load_gather(posv, [fr + comp])
                plsc.store_scatter(outp, [ro + comp], g)
        pltpu.sync_copy(outp, posd_hbm.at[pl.ds(wid * PW * 3, PW * 3)])

    return kfn(f2f, kidx2, fpsf, posf, scale, shift)


def kernel(pos, feat, cluster_idx, fps_idx, k_idx, w1a, b1a, w1b, b1b, w2a,
           b2a, w2b, b2b, wq, wk, wv, wo, bo, ln1_g, ln1_b, wtd, btd, bn_g,
           bn_b):
    B, N, _ = pos.shape
    ND = N // 4
    cid = cluster_idx.astype(jnp.int32)
    posT = jnp.transpose(pos, (0, 2, 1))

    cogg = _sc_cog(posT, cid).reshape(B, N, 4)

    w1a_p = jnp.concatenate([w1a, jnp.zeros((4, D), jnp.float32)], axis=0)
    w2a_p = jnp.concatenate([w2a[3:6], jnp.zeros((5, D), jnp.float32)],
                            axis=0)
    hpos, q, kv = _tc_rows(
        pos, feat, cogg, w1a_p, b1a.reshape(1, D), w1b, b1b.reshape(1, D),
        w2a_p, b2a.reshape(1, D), w2b, b2b.reshape(1, D), wq, wk, wv)

    NR = N // RB
    ar = jnp.arange(N, dtype=jnp.int32)[None, :]
    chg_lo = jnp.concatenate(
        [jnp.ones((B, 1), bool), cid[:, 1:] != cid[:, :-1]], axis=1)
    ss = lax.cummax(jnp.where(chg_lo, ar, 0), axis=1)
    endcand = jnp.where(
        jnp.concatenate([chg_lo[:, 1:], jnp.ones((B, 1), bool)], axis=1),
        ar + 1, N)
    se = lax.cummin(endcand[:, ::-1], axis=1)[:, ::-1]
    rowseg = jnp.stack(
        [ss.astype(jnp.float32), se.astype(jnp.float32)],
        axis=-1).reshape(B * NR, RB, 2)
    lo = (ss[:, ::RB] // CBK).astype(jnp.int32)
    nw = ((se[:, RB - 1::RB] - lo * CBK + WINC - 1) // WINC).astype(jnp.int32)

    f2, st = _tc_attn(q, kv, hpos, rowseg, lo, nw, wo,
                      bo.reshape(1, D), ln1_g.reshape(1, D),
                      ln1_b.reshape(1, D), wtd, btd.reshape(1, DOUT))

    cnt = B * N
    m = st[0] / cnt
    var = st[1] / cnt - m * m
    scale = bn_g / jnp.sqrt(var + 1e-5)
    shift = bn_b - m * scale

    boff = (jnp.arange(B, dtype=jnp.int32) * N)[:, None, None]
    kflat = (k_idx.astype(jnp.int32) + boff).reshape(-1, 128)
    fpsf = (fps_idx.astype(jnp.int32) + boff[:, :, 0]).reshape(-1)
    f2f = f2.reshape(B * N, DOUT)
    posf = pos.reshape(-1)

    featd, posd = _sc_down(f2f, kflat, fpsf, posf, scale, shift)
    pos_down = posd.reshape(B, ND, 3)
    feat_down = featd.reshape(B, ND, DOUT)
    return pos_down, feat_down
